# Initial kernel scaffold; baseline (speedup 1.0000x reference)
import jax
import jax.numpy as jnp
from jax.experimental import pallas as pl


def kernel(x, edge_index, W1, b1, W2, b2, W3, b3, Wm1, bm1, Wm2, bm2):
    raise NotImplementedError("write your pallas kernel here")


# speedup vs baseline: 10.2221x; 10.2221x over previous
import functools

import jax
import jax.numpy as jnp
from jax import lax
from jax.experimental import pallas as pl
from jax.experimental.pallas import tpu as pltpu
from jax.experimental.pallas import tpu_sc as plsc

_N = 10000
_D = 128
_E = 320000

_NC = 2
_NS = 16
_NW = _NC * _NS

_CH = 128
_K = 79
_EPAD = _NW * _K * _CH

_NACC = 10112
_RPT = _NACC // _NS
_NHIST = 10240
_HPT = _NHIST // _NS

_BR = 400


def _mesh():
    return plsc.VectorSubcoreMesh(core_axis_name="c", subcore_axis_name="s")


def _sc_degree(dstw):
    @functools.partial(
        pl.kernel,
        out_type=jax.ShapeDtypeStruct((_NC, _NHIST), jnp.float32),
        mesh=_mesh(),
        scratch_types=[
            pltpu.VMEM((_K, _CH), jnp.int32),
            pltpu.VMEM((_CH,), jnp.float32),
            pltpu.VMEM((_HPT,), jnp.float32),
            pltpu.VMEM_SHARED((_NHIST,), jnp.float32),
        ],
    )
    def kdeg(dst_hbm, out_hbm, dst_v, ones_v, zero_v, hist_sh):
        c = lax.axis_index("c")
        s = lax.axis_index("s")
        wid = c * _NS + s
        pltpu.sync_copy(dst_hbm.at[wid], dst_v)
        for t in range(_CH // 16):
            ones_v[pl.ds(t * 16, 16)] = jnp.full((16,), 1.0, jnp.float32)
        for t in range(_HPT // 16):
            zero_v[pl.ds(t * 16, 16)] = jnp.zeros((16,), jnp.float32)
        pltpu.sync_copy(zero_v, hist_sh.at[pl.ds(s * _HPT, _HPT)])
        plsc.subcore_barrier()

        def body(j, carry):
            pltpu.sync_copy(ones_v, hist_sh.at[dst_v.at[j]], add=True)
            return carry

        lax.fori_loop(0, _K, body, 0)
        plsc.subcore_barrier()
        pltpu.sync_copy(hist_sh.at[pl.ds(s * _HPT, _HPT)],
                        out_hbm.at[c, pl.ds(s * _HPT, _HPT)])

    return kdeg(dstw)


def _sc_scatter(g, srcw, dstw, zrows):
    @functools.partial(
        pl.kernel,
        out_type=jax.ShapeDtypeStruct((_NC, _NACC, _D), jnp.float32),
        mesh=_mesh(),
        scratch_types=[
            pltpu.VMEM((_K, _CH), jnp.int32),
            pltpu.VMEM((_K, _CH), jnp.int32),
            pltpu.VMEM((_CH, _D), jnp.float32),
            pltpu.VMEM_SHARED((_NACC, _D), jnp.float32),
            pltpu.SemaphoreType.DMA,
        ],
    )
    def kconv(g_hbm, src_hbm, dst_hbm, z_hbm, out_hbm,
              src_v, dst_v, buf, acc_sh, sem):
        c = lax.axis_index("c")
        s = lax.axis_index("s")
        wid = c * _NS + s
        pltpu.sync_copy(src_hbm.at[wid], src_v)
        pltpu.sync_copy(dst_hbm.at[wid], dst_v)
        pltpu.sync_copy(z_hbm.at[pl.ds(s * _RPT, _RPT)],
                        acc_sh.at[pl.ds(s * _RPT, _RPT)])
        plsc.subcore_barrier()

        def body(j, carry):
            pltpu.async_copy(g_hbm.at[src_v.at[j]], buf, sem).wait()
            pltpu.sync_copy(buf, acc_sh.at[dst_v.at[j]], add=True)
            return carry

        lax.fori_loop(0, _K, body, 0)
        plsc.subcore_barrier()
        pltpu.sync_copy(acc_sh.at[pl.ds(s * _RPT, _RPT)],
                        out_hbm.at[c, pl.ds(s * _RPT, _RPT)])

    return kconv(g, srcw, dstw, zrows)


def _tc_first(h0, h1, x, W1):
    def body(h0_ref, h1_ref, x_ref, w_ref, g_ref, dinv_ref):
        deg = h0_ref[...] + h1_ref[...] + 1.0
        dinv = lax.rsqrt(deg)
        dinv_ref[...] = dinv
        g_ref[...] = dinv * jnp.dot(x_ref[...], w_ref[...],
                                    preferred_element_type=jnp.float32)

    return pl.pallas_call(
        body,
        grid=(_N // _BR,),
        in_specs=[
            pl.BlockSpec((_BR, 1), lambda i: (i, 0)),
            pl.BlockSpec((_BR, 1), lambda i: (i, 0)),
            pl.BlockSpec((_BR, _D), lambda i: (i, 0)),
            pl.BlockSpec((_D, _D), lambda i: (0, 0)),
        ],
        out_specs=[
            pl.BlockSpec((_BR, _D), lambda i: (i, 0)),
            pl.BlockSpec((_BR, 1), lambda i: (i, 0)),
        ],
        out_shape=[
            jax.ShapeDtypeStruct((_N, _D), jnp.float32),
            jax.ShapeDtypeStruct((_N, 1), jnp.float32),
        ],
    )(h0, h1, x, W1)


def _tc_mid(acc, g, dinv, b, W):
    def body(a0_ref, a1_ref, g_ref, dinv_ref, b_ref, w_ref, out_ref):
        dinv = dinv_ref[...]
        h = jnp.maximum(
            dinv * (a0_ref[0] + a1_ref[0] + g_ref[...]) + b_ref[...], 0.0)
        out_ref[...] = dinv * jnp.dot(h, w_ref[...],
                                      preferred_element_type=jnp.float32)

    return pl.pallas_call(
        body,
        grid=(_N // _BR,),
        in_specs=[
            pl.BlockSpec((1, _BR, _D), lambda i: (0, i, 0)),
            pl.BlockSpec((1, _BR, _D), lambda i: (1, i, 0)),
            pl.BlockSpec((_BR, _D), lambda i: (i, 0)),
            pl.BlockSpec((_BR, 1), lambda i: (i, 0)),
            pl.BlockSpec((1, _D), lambda i: (0, 0)),
            pl.BlockSpec((_D, _D), lambda i: (0, 0)),
        ],
        out_specs=pl.BlockSpec((_BR, _D), lambda i: (i, 0)),
        out_shape=jax.ShapeDtypeStruct((_N, _D), jnp.float32),
    )(acc, acc, g, dinv, b, W)


def _tc_last(acc, g, dinv, b3, Wm1, bm1, Wm2, bm2):
    def body(a0_ref, a1_ref, g_ref, dinv_ref, b3_ref, wm1_ref, bm1_ref,
             wm2_ref, bm2_ref, out_ref):
        h3 = (dinv_ref[...] * (a0_ref[0] + a1_ref[0] + g_ref[...])
              + b3_ref[...])
        m = jnp.maximum(
            jnp.dot(h3, wm1_ref[...], preferred_element_type=jnp.float32)
            + bm1_ref[...], 0.0)
        out_ref[...] = (jnp.dot(m, wm2_ref[...],
                                preferred_element_type=jnp.float32)
                        + bm2_ref[...])

    return pl.pallas_call(
        body,
        grid=(_N // _BR,),
        in_specs=[
            pl.BlockSpec((1, _BR, _D), lambda i: (0, i, 0)),
            pl.BlockSpec((1, _BR, _D), lambda i: (1, i, 0)),
            pl.BlockSpec((_BR, _D), lambda i: (i, 0)),
            pl.BlockSpec((_BR, 1), lambda i: (i, 0)),
            pl.BlockSpec((1, _D), lambda i: (0, 0)),
            pl.BlockSpec((_D, _D), lambda i: (0, 0)),
            pl.BlockSpec((1, _D), lambda i: (0, 0)),
            pl.BlockSpec((_D, 1), lambda i: (0, 0)),
            pl.BlockSpec((1, 1), lambda i: (0, 0)),
        ],
        out_specs=pl.BlockSpec((_BR, 1), lambda i: (i, 0)),
        out_shape=jax.ShapeDtypeStruct((_N, 1), jnp.float32),
    )(acc, acc, g, dinv, b3, Wm1, bm1, Wm2, bm2)


def kernel(x, edge_index, W1, b1, W2, b2, W3, b3, Wm1, bm1, Wm2, bm2):
    src = edge_index[0]
    dst = edge_index[1]
    pad = _EPAD - _E
    srcw = jnp.concatenate(
        [src, jnp.zeros((pad,), jnp.int32)]).reshape(_NW, _K, _CH)
    dstw = jnp.concatenate(
        [dst, jnp.full((pad,), _N, jnp.int32)]).reshape(_NW, _K, _CH)
    zrows = jnp.zeros((_NACC, _D), jnp.float32)

    hist = _sc_degree(dstw)
    h0 = hist[0, :_N].reshape(_N, 1)
    h1 = hist[1, :_N].reshape(_N, 1)

    g1, dinv = _tc_first(h0, h1, x, W1)
    acc1 = _sc_scatter(g1, srcw, dstw, zrows)
    g2 = _tc_mid(acc1, g1, dinv, b1.reshape(1, _D), W2)
    acc2 = _sc_scatter(g2, srcw, dstw, zrows)
    g3 = _tc_mid(acc2, g2, dinv, b2.reshape(1, _D), W3)
    acc3 = _sc_scatter(g3, srcw, dstw, zrows)
    out = _tc_last(acc3, g3, dinv, b3.reshape(1, _D), Wm1,
                   bm1.reshape(1, _D), Wm2, bm2.reshape(1, 1))
    return out
